# baseline (speedup 1.0000x reference)
import jax
import jax.numpy as jnp
import numpy as np
from jax import lax
from jax.experimental import pallas as pl
from jax.experimental.pallas import tpu as pltpu

_BN_EPS = 1e-5
_LANES = 128
_TB = (((1,), (1,)), ((), ()))


def _build_colup_matrix(W):
    W2 = 2 * W
    S = np.zeros((3 * W2, W + 2), np.float32)
    for dx in range(3):
        for c in range(W2):
            u = c - 1 + dx
            if 0 <= u < W2:
                j = u // 2
                if u % 2 == 0:
                    S[dx * W2 + c, j] += 0.25
                    S[dx * W2 + c, j + 1] += 0.75
                else:
                    S[dx * W2 + c, j + 1] += 0.75
                    S[dx * W2 + c, j + 2] += 0.25
    return jnp.asarray(S, jnp.bfloat16)


def _pick_row_tile(h):
    for cand in (16, 12, 8, 6, 4, 3, 2, 1):
        if h % cand == 0:
            return cand
    return 1


def _make_pass1_body(TR, W, Cin, Cpad):
    TR2 = 2 * TR
    R2 = TR2 + 2
    W2 = 2 * W

    def body(x_ref, s_ref, w_ref, y_ref, stats_ref, acc_ref):
        t = pl.program_id(1)
        t_last = pl.num_programs(1) - 1

        xb = x_ref[...]

        c34 = jnp.bfloat16(0.75)
        c14 = jnp.bfloat16(0.25)
        lo = xb[:-1]
        hi = xb[1:]
        ev = lo * c34 + hi * c14
        od = lo * c14 + hi * c34
        ru = jnp.stack([ev, od], axis=1).reshape(R2, W + 2, Cin)

        row = lax.broadcasted_iota(jnp.int32, (R2, 1, 1), 0)
        edge = jnp.logical_or(jnp.logical_and(t == 0, row == 0),
                              jnp.logical_and(t == t_last, row == R2 - 1))
        ru = jnp.where(edge, jnp.bfloat16(0.0), ru)

        sm = s_ref[...]
        rows = [lax.dot_general(sm, ru[r], (((1,), (0,)), ((), ())),
                                preferred_element_type=jnp.float32
                                ).astype(jnp.bfloat16)
                for r in range(R2)]
        wb = jnp.stack(rows, axis=0)

        P2 = TR2 * W2
        acc = jnp.zeros((Cpad, P2), jnp.float32)
        for dy in range(3):
            for dx in range(3):
                win = wb[dy:dy + TR2, dx * W2:(dx + 1) * W2]
                acc += lax.dot_general(w_ref[dy * 3 + dx],
                                       win.reshape(P2, Cin), _TB,
                                       preferred_element_type=jnp.float32)

        @pl.when(t == 0)
        def _():
            acc_ref[...] = jnp.zeros_like(acc_ref)

        acc_ref[:, 0:1] += jnp.sum(acc, axis=1, keepdims=True)
        acc_ref[:, 1:2] += jnp.sum(acc * acc, axis=1, keepdims=True)

        @pl.when(t == t_last)
        def _():
            stats_ref[...] = acc_ref[...]

        y_ref[...] = acc.astype(jnp.bfloat16)

    return body


def _bn_act_body(y_ref, scale_ref, shift_ref, a_ref, o_ref):
    z = y_ref[...].astype(jnp.float32)
    z = z * scale_ref[...] + shift_ref[...]
    slope = a_ref[0]
    o_ref[...] = jnp.where(z > 0, z, slope * z)


@jax.jit
def _forward(x_nchw, conv_w, conv_b, bn_gamma, bn_beta, prelu_a):
    del conv_b
    N, Cin, H, W = x_nchw.shape
    Cout = conv_w.shape[0]
    TR = _pick_row_tile(H)
    T = H // TR
    TR2 = 2 * TR
    P = TR2 * W
    Cpad = ((Cout + _LANES - 1) // _LANES) * _LANES

    x_nhwc = jnp.transpose(x_nchw.astype(jnp.bfloat16), (0, 2, 3, 1))
    xp = jnp.pad(x_nhwc, ((0, 0), (1, 1), (1, 1), (0, 0)), mode="edge")
    xt = jnp.stack([xp[:, t * TR:t * TR + TR + 2] for t in range(T)], axis=1)

    w9 = jnp.transpose(conv_w, (2, 3, 0, 1)).reshape(9, Cout, Cin)
    wp = jnp.pad(w9, ((0, 0), (0, Cpad - Cout), (0, 0))).astype(jnp.bfloat16)
    smat = _build_colup_matrix(W)

    y, stats = pl.pallas_call(
        _make_pass1_body(TR, W, Cin, Cpad),
        out_shape=(
            jax.ShapeDtypeStruct((N, Cpad, 4 * H * W), jnp.bfloat16),
            jax.ShapeDtypeStruct((N, Cpad, 8), jnp.float32),
        ),
        grid=(N, T),
        in_specs=[
            pl.BlockSpec((None, None, TR + 2, W + 2, Cin),
                         lambda n, t: (n, t, 0, 0, 0)),
            pl.BlockSpec((6 * W, W + 2), lambda n, t: (0, 0)),
            pl.BlockSpec((9, Cpad, Cin), lambda n, t: (0, 0, 0)),
        ],
        out_specs=(
            pl.BlockSpec((None, Cpad, 2 * P), lambda n, t: (n, 0, t)),
            pl.BlockSpec((None, Cpad, 8), lambda n, t: (n, 0, 0)),
        ),
        scratch_shapes=[pltpu.VMEM((Cpad, 8), jnp.float32)],
        compiler_params=pltpu.CompilerParams(
            dimension_semantics=("parallel", "arbitrary")),
    )(xt, smat, wp)

    stot = jnp.sum(stats, axis=0)
    m_total = jnp.float32(N * (2 * H) * (2 * W))
    mean = stot[:, 0] / m_total
    var = jnp.maximum(stot[:, 1] / m_total - mean * mean, 0.0)
    gamma_p = jnp.pad(bn_gamma.astype(jnp.float32), (0, Cpad - Cout))
    beta_p = jnp.pad(bn_beta.astype(jnp.float32), (0, Cpad - Cout))
    scale = (gamma_p * lax.rsqrt(var + _BN_EPS)).reshape(Cpad, 1)
    shift = (beta_p.reshape(Cpad, 1) - mean.reshape(Cpad, 1) * scale)
    a_smem = prelu_a.reshape(1).astype(jnp.float32)

    out = pl.pallas_call(
        _bn_act_body,
        out_shape=jax.ShapeDtypeStruct((N, Cpad, 4 * H * W), jnp.float32),
        grid=(N, T),
        in_specs=[
            pl.BlockSpec((None, Cpad, 2 * P), lambda n, t: (n, 0, t)),
            pl.BlockSpec((Cpad, 1), lambda n, t: (0, 0)),
            pl.BlockSpec((Cpad, 1), lambda n, t: (0, 0)),
            pl.BlockSpec(memory_space=pltpu.MemorySpace.SMEM),
        ],
        out_specs=pl.BlockSpec((None, Cpad, 2 * P), lambda n, t: (n, 0, t)),
        compiler_params=pltpu.CompilerParams(
            dimension_semantics=("parallel", "parallel")),
    )(y, scale, shift, a_smem)

    return out[:, :Cout].reshape(N, Cout, 2 * H, 2 * W)


def kernel(x_nchw, conv_w, conv_b, bn_gamma, bn_beta, prelu_a):
    return _forward(x_nchw, conv_w, conv_b, bn_gamma, bn_beta, prelu_a)


# speedup vs baseline: 1.1804x; 1.1804x over previous
import jax
import jax.numpy as jnp
import numpy as np
from jax import lax
from jax.experimental import pallas as pl
from jax.experimental.pallas import tpu as pltpu

_BN_EPS = 1e-5
_LANES = 128
_TB = (((1,), (1,)), ((), ()))


def _build_colup_matrix(W):
    W2 = 2 * W
    S = np.zeros((3 * W2, W), np.float32)
    for dx in range(3):
        for c in range(W2):
            u = c - 1 + dx
            if 0 <= u < W2:
                j = u // 2
                if u % 2 == 0:
                    S[dx * W2 + c, max(j - 1, 0)] += 0.25
                    S[dx * W2 + c, j] += 0.75
                else:
                    S[dx * W2 + c, j] += 0.75
                    S[dx * W2 + c, min(j + 1, W - 1)] += 0.25
    return jnp.asarray(S, jnp.bfloat16)


def _pick_row_tile(h):
    for cand in (24, 16, 12, 8, 6, 4, 2):
        if h % cand == 0:
            return cand
    return 1


def _make_pass1_body(TR, W, Cin, Cpad):
    TR2 = 2 * TR
    R2 = TR2 + 2
    W2 = 2 * W

    def body(xm_ref, xt_ref, xb_ref, s_ref, w_ref, y_ref, stats_ref, acc_ref):
        t = pl.program_id(1)
        t_last = pl.num_programs(1) - 1

        top = jnp.where(t == 0, xm_ref[0], xt_ref[1])
        bot = jnp.where(t == t_last, xm_ref[TR - 1], xb_ref[0])
        src = [top] + [xm_ref[r] for r in range(TR)] + [bot]

        c34 = jnp.bfloat16(0.75)
        c14 = jnp.bfloat16(0.25)
        up = []
        for i in range(TR + 1):
            a, b = src[i], src[i + 1]
            up.append(a * c34 + b * c14)
            up.append(a * c14 + b * c34)
        up[0] = up[0] * (t > 0).astype(jnp.bfloat16)
        up[-1] = up[-1] * (t < t_last).astype(jnp.bfloat16)

        sm = s_ref[...]
        rows = []
        for k in range(R2 // 2):
            pair = jnp.concatenate([up[2 * k], up[2 * k + 1]], axis=1)
            pr = lax.dot_general(sm, pair, (((1,), (0,)), ((), ())),
                                 preferred_element_type=jnp.float32)
            rows.append(pr[:, :Cin].astype(jnp.bfloat16))
            rows.append(pr[:, Cin:].astype(jnp.bfloat16))
        wb = jnp.stack(rows, axis=0)

        P2 = TR2 * W2
        acc = jnp.zeros((Cpad, P2), jnp.float32)
        for dy in range(3):
            for dx in range(3):
                win = wb[dy:dy + TR2, dx * W2:(dx + 1) * W2]
                acc += lax.dot_general(w_ref[dy * 3 + dx],
                                       win.reshape(P2, Cin), _TB,
                                       preferred_element_type=jnp.float32)

        @pl.when(t == 0)
        def _():
            acc_ref[...] = jnp.zeros_like(acc_ref)

        acc_ref[:, 0:1] += jnp.sum(acc, axis=1, keepdims=True)
        acc_ref[:, 1:2] += jnp.sum(acc * acc, axis=1, keepdims=True)

        @pl.when(t == t_last)
        def _():
            stats_ref[...] = acc_ref[...]

        y_ref[...] = acc.astype(jnp.bfloat16)

    return body


def _bn_act_body(y_ref, scale_ref, shift_ref, a_ref, o_ref):
    z = y_ref[...].astype(jnp.float32)
    z = z * scale_ref[...] + shift_ref[...]
    slope = a_ref[0]
    o_ref[...] = jnp.where(z > 0, z, slope * z)


@jax.jit
def _forward(x_nchw, conv_w, conv_b, bn_gamma, bn_beta, prelu_a):
    del conv_b
    N, Cin, H, W = x_nchw.shape
    Cout = conv_w.shape[0]
    TR = _pick_row_tile(H)
    T = H // TR
    TR2 = 2 * TR
    P = TR2 * W
    HB = TR // 2
    Cpad = ((Cout + _LANES - 1) // _LANES) * _LANES

    x_nhwc = jnp.transpose(x_nchw.astype(jnp.bfloat16), (0, 2, 3, 1))

    w9 = jnp.transpose(conv_w, (2, 3, 0, 1)).reshape(9, Cout, Cin)
    wp = jnp.pad(w9, ((0, 0), (0, Cpad - Cout), (0, 0))).astype(jnp.bfloat16)
    smat = _build_colup_matrix(W)

    y, stats = pl.pallas_call(
        _make_pass1_body(TR, W, Cin, Cpad),
        out_shape=(
            jax.ShapeDtypeStruct((N, Cpad, 4 * H * W), jnp.bfloat16),
            jax.ShapeDtypeStruct((N, Cpad, 8), jnp.float32),
        ),
        grid=(N, T),
        in_specs=[
            pl.BlockSpec((None, TR, W, Cin), lambda n, t: (n, t, 0, 0)),
            pl.BlockSpec((None, 2, W, Cin),
                         lambda n, t: (n, jnp.maximum(t * HB - 1, 0), 0, 0)),
            pl.BlockSpec((None, 2, W, Cin),
                         lambda n, t: (n, jnp.minimum(t * HB + HB,
                                                      H // 2 - 1), 0, 0)),
            pl.BlockSpec((6 * W, W), lambda n, t: (0, 0)),
            pl.BlockSpec((9, Cpad, Cin), lambda n, t: (0, 0, 0)),
        ],
        out_specs=(
            pl.BlockSpec((None, Cpad, 2 * P), lambda n, t: (n, 0, t)),
            pl.BlockSpec((None, Cpad, 8), lambda n, t: (n, 0, 0)),
        ),
        scratch_shapes=[pltpu.VMEM((Cpad, 8), jnp.float32)],
        compiler_params=pltpu.CompilerParams(
            dimension_semantics=("parallel", "arbitrary")),
    )(x_nhwc, x_nhwc, x_nhwc, smat, wp)

    stot = jnp.sum(stats, axis=0)
    m_total = jnp.float32(N * (2 * H) * (2 * W))
    mean = stot[:, 0] / m_total
    var = jnp.maximum(stot[:, 1] / m_total - mean * mean, 0.0)
    gamma_p = jnp.pad(bn_gamma.astype(jnp.float32), (0, Cpad - Cout))
    beta_p = jnp.pad(bn_beta.astype(jnp.float32), (0, Cpad - Cout))
    scale = (gamma_p * lax.rsqrt(var + _BN_EPS)).reshape(Cpad, 1)
    shift = (beta_p.reshape(Cpad, 1) - mean.reshape(Cpad, 1) * scale)
    a_smem = prelu_a.reshape(1).astype(jnp.float32)

    out = pl.pallas_call(
        _bn_act_body,
        out_shape=jax.ShapeDtypeStruct((N, Cpad, 4 * H * W), jnp.float32),
        grid=(N, T),
        in_specs=[
            pl.BlockSpec((None, Cpad, 2 * P), lambda n, t: (n, 0, t)),
            pl.BlockSpec((Cpad, 1), lambda n, t: (0, 0)),
            pl.BlockSpec((Cpad, 1), lambda n, t: (0, 0)),
            pl.BlockSpec(memory_space=pltpu.MemorySpace.SMEM),
        ],
        out_specs=pl.BlockSpec((None, Cpad, 2 * P), lambda n, t: (n, 0, t)),
        compiler_params=pltpu.CompilerParams(
            dimension_semantics=("parallel", "parallel")),
    )(y, scale, shift, a_smem)

    return out[:, :Cout].reshape(N, Cout, 2 * H, 2 * W)


def kernel(x_nchw, conv_w, conv_b, bn_gamma, bn_beta, prelu_a):
    return _forward(x_nchw, conv_w, conv_b, bn_gamma, bn_beta, prelu_a)
